# baseline (speedup 1.0000x reference)
import jax
import jax.numpy as jnp
from jax import lax
from jax.experimental import pallas as pl
from jax.experimental.pallas import tpu as pltpu
from jax.experimental.pallas import tpu_sc as plsc

D = 128
DW = 16
CH = 128
NC = 2
NS = 16
NW = NC * NS
L = 16
GW = 20

_SC_PARAMS = pltpu.CompilerParams(
    use_tc_tiling_on_sc=False, needs_layout_passes=False)


def _msg_body(nf, w1, b1, w2, b2, out):
    h = jnp.maximum(jnp.dot(nf[...], w1[...], preferred_element_type=jnp.float32) + b1[...], 0.0)
    out[...] = jnp.dot(h, w2[...], preferred_element_type=jnp.float32) + b2[...]


def _sc_deg_body(nchunk, idxr, zr, degout, dst_v, w_v, deg_src, deg_sh, sem):
    cid = lax.axis_index("c")
    sid = lax.axis_index("s")
    wid = cid * NS + sid
    n_pad = deg_sh.shape[0]
    rpt = n_pad // NS

    pltpu.sync_copy(idxr.at[1, wid], dst_v)
    pltpu.sync_copy(idxr.at[2, wid], w_v)
    pltpu.sync_copy(zr.at[pl.ds(sid * rpt, rpt)], deg_sh.at[pl.ds(sid * rpt, rpt)])

    def zrow(i, c):
        deg_src[i, pl.ds(0, L)] = jnp.zeros((L,), jnp.float32)
        return c
    lax.fori_loop(0, CH, zrow, 0)
    plsc.subcore_barrier()

    col0 = jnp.zeros((L,), jnp.int32)
    iota = lax.broadcasted_iota(jnp.int32, (L,), 0)

    def chunk(j, carry):
        for g in range(CH // L):
            w16 = plsc.bitcast(w_v[j, pl.ds(g * L, L)], jnp.float32)
            plsc.store_scatter(deg_src, [g * L + iota, col0], w16)
        pltpu.sync_copy(deg_src, deg_sh.at[dst_v.at[j]], add=True)
        return carry

    lax.fori_loop(0, nchunk, chunk, 0)
    plsc.subcore_barrier()
    pltpu.sync_copy(deg_sh.at[pl.ds(sid * rpt, rpt)],
                    degout.at[cid, pl.ds(sid * rpt, rpt)])


def _sc_agg_body(nchunk, msgs, idxr, zr, aggout,
                 src_v, dst_v, w_v, rows_v, agg_sh, sem_g):
    cid = lax.axis_index("c")
    sid = lax.axis_index("s")
    wid = cid * NS + sid
    n_pad = agg_sh.shape[0]
    rpt = n_pad // NS

    pltpu.sync_copy(zr.at[pl.ds(sid * rpt, rpt)], agg_sh.at[pl.ds(sid * rpt, rpt)])
    plsc.subcore_barrier()

    ngrp = nchunk // GW

    def group(g, carry):
        base = g * GW
        pltpu.sync_copy(idxr.at[0, wid, pl.ds(base, GW)], src_v)
        pltpu.sync_copy(idxr.at[1, wid, pl.ds(base, GW)], dst_v)
        pltpu.sync_copy(idxr.at[2, wid, pl.ds(base, GW)], w_v)

        def chunk(j, c0):
            pltpu.async_copy(msgs.at[src_v.at[j]], rows_v, sem_g).wait()
            jv = jnp.full((L,), j, jnp.int32)

            @plsc.parallel_loop(0, CH, step=1, unroll=8)
            def row(i):
                ws = plsc.bitcast(plsc.load_gather(
                    w_v, [jv, jnp.full((L,), i, jnp.int32)]), jnp.float32)
                for c in range(D // L):
                    rows_v[i, pl.ds(c * L, L)] = rows_v[i, pl.ds(c * L, L)] * ws

            pltpu.sync_copy(rows_v, agg_sh.at[dst_v.at[j]], add=True)
            return c0

        lax.fori_loop(0, GW, chunk, 0)
        return carry

    lax.fori_loop(0, ngrp, group, 0)
    plsc.subcore_barrier()
    pltpu.sync_copy(agg_sh.at[pl.ds(sid * rpt, rpt)],
                    aggout.at[cid, pl.ds(sid * rpt, rpt)])


def _final_body(aggp, degp, nf, wga, wgn, bg, wua, wun, bu1, wu2, bu2, gamma, beta, out):
    p = aggp[...]
    dp = degp[...]
    deg = jnp.maximum((dp[0] + dp[1])[:, 0:1], 1.0)
    agg = (p[0] + p[1]) / deg
    x = nf[...]
    gate = jax.nn.sigmoid(
        jnp.dot(agg, wga[...], preferred_element_type=jnp.float32)
        + jnp.dot(x, wgn[...], preferred_element_type=jnp.float32) + bg[...])
    u = jnp.maximum(
        jnp.dot(agg, wua[...], preferred_element_type=jnp.float32)
        + jnp.dot(x, wun[...], preferred_element_type=jnp.float32) + bu1[...], 0.0)
    upd = jnp.dot(u, wu2[...], preferred_element_type=jnp.float32) + bu2[...]
    o = gate * upd + (1.0 - gate) * x
    mean = jnp.mean(o, axis=-1, keepdims=True)
    var = jnp.mean((o - mean) ** 2, axis=-1, keepdims=True)
    out[...] = (o - mean) / jnp.sqrt(var + 1e-3) * gamma[...] + beta[...]


def kernel(node_feats, neighbor_feats, edge_indices, edge_weights,
           W1, b1, W2, b2, Wg, bg, Wu1, bu1, Wu2, bu2, gamma, beta):
    n = node_feats.shape[0]
    e = edge_indices.shape[1]
    bn = 1000
    grid = (n // bn,)

    full = lambda i: (0, 0)
    row_spec = pl.BlockSpec((bn, D), lambda i: (i, 0))
    w_spec = pl.BlockSpec((D, D), full)
    b_spec = pl.BlockSpec((1, D), full)

    msgs = pl.pallas_call(
        _msg_body,
        grid=grid,
        in_specs=[row_spec, w_spec, b_spec, w_spec, b_spec],
        out_specs=row_spec,
        out_shape=jax.ShapeDtypeStruct((n, D), jnp.float32),
    )(neighbor_feats, W1, b1.reshape(1, D), W2, b2.reshape(1, D))

    epw = -(-e // (NW * GW * CH)) * (GW * CH)
    ep = epw * NW
    pad = ep - e
    src = jnp.concatenate([edge_indices[0], jnp.zeros((pad,), jnp.int32)])
    dst = jnp.concatenate([edge_indices[1], jnp.zeros((pad,), jnp.int32)])
    w = jnp.concatenate([edge_weights[:, 0], jnp.zeros((pad,), jnp.float32)])
    nchunk = epw // CH
    idx3 = jnp.stack([
        src.reshape(NW, nchunk, CH),
        dst.reshape(NW, nchunk, CH),
        lax.bitcast_convert_type(w, jnp.int32).reshape(NW, nchunk, CH),
    ])
    n_pad = -(-n // (NS * 8)) * (NS * 8)

    deg_partial = pl.kernel(
        lambda *refs: _sc_deg_body(nchunk, *refs),
        out_type=jax.ShapeDtypeStruct((NC, n_pad, DW), jnp.float32),
        mesh=plsc.VectorSubcoreMesh(
            core_axis_name="c", subcore_axis_name="s",
            num_cores=NC, num_subcores=NS),
        scratch_types=[
            pltpu.VMEM((nchunk, CH), jnp.int32),
            pltpu.VMEM((nchunk, CH), jnp.int32),
            pltpu.VMEM((CH, DW), jnp.float32),
            pltpu.VMEM_SHARED((n_pad, DW), jnp.float32),
            pltpu.SemaphoreType.DMA,
        ],
        compiler_params=_SC_PARAMS,
    )(idx3, jnp.zeros((n_pad, DW), jnp.float32))

    agg_partial = pl.kernel(
        lambda *refs: _sc_agg_body(nchunk, *refs),
        out_type=jax.ShapeDtypeStruct((NC, n_pad, D), jnp.float32),
        mesh=plsc.VectorSubcoreMesh(
            core_axis_name="c", subcore_axis_name="s",
            num_cores=NC, num_subcores=NS),
        scratch_types=[
            pltpu.VMEM((GW, CH), jnp.int32),
            pltpu.VMEM((GW, CH), jnp.int32),
            pltpu.VMEM((GW, CH), jnp.int32),
            pltpu.VMEM((CH, D), jnp.float32),
            pltpu.VMEM_SHARED((n_pad, D), jnp.float32),
            pltpu.SemaphoreType.DMA,
        ],
        compiler_params=_SC_PARAMS,
    )(msgs, idx3, jnp.zeros((n_pad, D), jnp.float32))

    out = pl.pallas_call(
        _final_body,
        grid=grid,
        in_specs=[pl.BlockSpec((NC, bn, D), lambda i: (0, i, 0)),
                  pl.BlockSpec((NC, bn, DW), lambda i: (0, i, 0)),
                  row_spec, w_spec, w_spec, b_spec,
                  w_spec, w_spec, b_spec, w_spec, b_spec, b_spec, b_spec],
        out_specs=row_spec,
        out_shape=jax.ShapeDtypeStruct((n, D), jnp.float32),
    )(agg_partial, deg_partial, node_feats, Wg[:D], Wg[D:], bg.reshape(1, D),
      Wu1[:D], Wu1[D:], bu1.reshape(1, D), Wu2, bu2.reshape(1, D),
      gamma.reshape(1, D), beta.reshape(1, D))
    return out


# speedup vs baseline: 1.2832x; 1.2832x over previous
import jax
import jax.numpy as jnp
from jax import lax
from jax.experimental import pallas as pl
from jax.experimental.pallas import tpu as pltpu
from jax.experimental.pallas import tpu_sc as plsc

D = 128
DW = 16
CH = 128
NC = 2
NS = 16
NW = NC * NS
L = 16
GW = 20

_SC_PARAMS = pltpu.CompilerParams(
    use_tc_tiling_on_sc=False, needs_layout_passes=False)


def _msg_body(nf, w1, b1, w2, b2, out):
    h = jnp.maximum(jnp.dot(nf[...], w1[...], preferred_element_type=jnp.float32) + b1[...], 0.0)
    out[...] = jnp.dot(h, w2[...], preferred_element_type=jnp.float32) + b2[...]


def _sc_deg_body(nchunk, idxr, zr, degout, dst_v, w_v, deg_src, deg_sh, sem):
    cid = lax.axis_index("c")
    sid = lax.axis_index("s")
    wid = cid * NS + sid
    n_pad = deg_sh.shape[0]
    rpt = n_pad // NS

    pltpu.sync_copy(idxr.at[1, wid], dst_v)
    pltpu.sync_copy(idxr.at[2, wid], w_v)
    pltpu.sync_copy(zr.at[pl.ds(sid * rpt, rpt)], deg_sh.at[pl.ds(sid * rpt, rpt)])

    def zrow(i, c):
        deg_src[i, pl.ds(0, L)] = jnp.zeros((L,), jnp.float32)
        return c
    lax.fori_loop(0, CH, zrow, 0)
    plsc.subcore_barrier()

    col0 = jnp.zeros((L,), jnp.int32)
    iota = lax.broadcasted_iota(jnp.int32, (L,), 0)

    def chunk(j, carry):
        for g in range(CH // L):
            w16 = plsc.bitcast(w_v[j, pl.ds(g * L, L)], jnp.float32)
            plsc.store_scatter(deg_src, [g * L + iota, col0], w16)
        pltpu.sync_copy(deg_src, deg_sh.at[dst_v.at[j]], add=True)
        return carry

    lax.fori_loop(0, nchunk, chunk, 0)
    plsc.subcore_barrier()
    pltpu.sync_copy(deg_sh.at[pl.ds(sid * rpt, rpt)],
                    degout.at[cid, pl.ds(sid * rpt, rpt)])


def _sc_agg_body(nchunk, msgs, idxr, zr, aggout,
                 src_v, dst_v, w_v, rows_v, agg_sh, sem_g):
    cid = lax.axis_index("c")
    sid = lax.axis_index("s")
    wid = cid * NS + sid
    n_pad = agg_sh.shape[0]
    rpt = n_pad // NS

    pltpu.sync_copy(zr.at[pl.ds(sid * rpt, rpt)], agg_sh.at[pl.ds(sid * rpt, rpt)])
    plsc.subcore_barrier()

    pltpu.sync_copy(idxr.at[0, wid], src_v)
    pltpu.sync_copy(idxr.at[1, wid], dst_v)
    pltpu.sync_copy(idxr.at[2, wid], w_v)

    def chunk(j, c0):
        pltpu.async_copy(msgs.at[src_v.at[j]], rows_v, sem_g).wait()
        jv = jnp.full((L,), j, jnp.int32)

        def row(i, c2):
            ws = plsc.bitcast(plsc.load_gather(
                w_v, [jv, jnp.full((L,), i, jnp.int32)]), jnp.float32)
            for c in range(D // L):
                rows_v[i, pl.ds(c * L, L)] = rows_v[i, pl.ds(c * L, L)] * ws
            return c2

        lax.fori_loop(0, CH, row, 0)
        pltpu.sync_copy(rows_v, agg_sh.at[dst_v.at[j]], add=True)
        return c0

    lax.fori_loop(0, nchunk, chunk, 0)
    plsc.subcore_barrier()
    pltpu.sync_copy(agg_sh.at[pl.ds(sid * rpt, rpt)],
                    aggout.at[cid, pl.ds(sid * rpt, rpt)])


def _final_body(aggp, degp, nf, wga, wgn, bg, wua, wun, bu1, wu2, bu2, gamma, beta, out):
    p = aggp[...]
    dp = degp[...]
    deg = jnp.maximum((dp[0] + dp[1])[:, 0:1], 1.0)
    agg = (p[0] + p[1]) / deg
    x = nf[...]
    gate = jax.nn.sigmoid(
        jnp.dot(agg, wga[...], preferred_element_type=jnp.float32)
        + jnp.dot(x, wgn[...], preferred_element_type=jnp.float32) + bg[...])
    u = jnp.maximum(
        jnp.dot(agg, wua[...], preferred_element_type=jnp.float32)
        + jnp.dot(x, wun[...], preferred_element_type=jnp.float32) + bu1[...], 0.0)
    upd = jnp.dot(u, wu2[...], preferred_element_type=jnp.float32) + bu2[...]
    o = gate * upd + (1.0 - gate) * x
    mean = jnp.mean(o, axis=-1, keepdims=True)
    var = jnp.mean((o - mean) ** 2, axis=-1, keepdims=True)
    out[...] = (o - mean) / jnp.sqrt(var + 1e-3) * gamma[...] + beta[...]


def kernel(node_feats, neighbor_feats, edge_indices, edge_weights,
           W1, b1, W2, b2, Wg, bg, Wu1, bu1, Wu2, bu2, gamma, beta):
    n = node_feats.shape[0]
    e = edge_indices.shape[1]
    bn = 1000
    grid = (n // bn,)

    full = lambda i: (0, 0)
    row_spec = pl.BlockSpec((bn, D), lambda i: (i, 0))
    w_spec = pl.BlockSpec((D, D), full)
    b_spec = pl.BlockSpec((1, D), full)

    msgs = pl.pallas_call(
        _msg_body,
        grid=grid,
        in_specs=[row_spec, w_spec, b_spec, w_spec, b_spec],
        out_specs=row_spec,
        out_shape=jax.ShapeDtypeStruct((n, D), jnp.float32),
    )(neighbor_feats, W1, b1.reshape(1, D), W2, b2.reshape(1, D))

    epw = -(-e // (NW * CH)) * CH
    ep = epw * NW
    pad = ep - e
    src = jnp.concatenate([edge_indices[0], jnp.zeros((pad,), jnp.int32)])
    dst = jnp.concatenate([edge_indices[1], jnp.zeros((pad,), jnp.int32)])
    w = jnp.concatenate([edge_weights[:, 0], jnp.zeros((pad,), jnp.float32)])
    nchunk = epw // CH
    idx3 = jnp.stack([
        src.reshape(NW, nchunk, CH),
        dst.reshape(NW, nchunk, CH),
        lax.bitcast_convert_type(w, jnp.int32).reshape(NW, nchunk, CH),
    ])
    n_pad = -(-n // (NS * 8)) * (NS * 8)

    deg_partial = pl.kernel(
        lambda *refs: _sc_deg_body(nchunk, *refs),
        out_type=jax.ShapeDtypeStruct((NC, n_pad, DW), jnp.float32),
        mesh=plsc.VectorSubcoreMesh(
            core_axis_name="c", subcore_axis_name="s",
            num_cores=NC, num_subcores=NS),
        scratch_types=[
            pltpu.VMEM((nchunk, CH), jnp.int32),
            pltpu.VMEM((nchunk, CH), jnp.int32),
            pltpu.VMEM((CH, DW), jnp.float32),
            pltpu.VMEM_SHARED((n_pad, DW), jnp.float32),
            pltpu.SemaphoreType.DMA,
        ],
        compiler_params=_SC_PARAMS,
    )(idx3, jnp.zeros((n_pad, DW), jnp.float32))

    agg_partial = pl.kernel(
        lambda *refs: _sc_agg_body(nchunk, *refs),
        out_type=jax.ShapeDtypeStruct((NC, n_pad, D), jnp.float32),
        mesh=plsc.VectorSubcoreMesh(
            core_axis_name="c", subcore_axis_name="s",
            num_cores=NC, num_subcores=NS),
        scratch_types=[
            pltpu.VMEM((nchunk, CH), jnp.int32),
            pltpu.VMEM((nchunk, CH), jnp.int32),
            pltpu.VMEM((nchunk, CH), jnp.int32),
            pltpu.VMEM((CH, D), jnp.float32),
            pltpu.VMEM_SHARED((n_pad, D), jnp.float32),
            pltpu.SemaphoreType.DMA,
        ],
        compiler_params=_SC_PARAMS,
    )(msgs, idx3, jnp.zeros((n_pad, D), jnp.float32))

    out = pl.pallas_call(
        _final_body,
        grid=grid,
        in_specs=[pl.BlockSpec((NC, bn, D), lambda i: (0, i, 0)),
                  pl.BlockSpec((NC, bn, DW), lambda i: (0, i, 0)),
                  row_spec, w_spec, w_spec, b_spec,
                  w_spec, w_spec, b_spec, w_spec, b_spec, b_spec, b_spec],
        out_specs=row_spec,
        out_shape=jax.ShapeDtypeStruct((n, D), jnp.float32),
    )(agg_partial, deg_partial, node_feats, Wg[:D], Wg[D:], bg.reshape(1, D),
      Wu1[:D], Wu1[D:], bu1.reshape(1, D), Wu2, bu2.reshape(1, D),
      gamma.reshape(1, D), beta.reshape(1, D))
    return out


# speedup vs baseline: 1.2851x; 1.0015x over previous
import jax
import jax.numpy as jnp
from jax import lax
from jax.experimental import pallas as pl
from jax.experimental.pallas import tpu as pltpu
from jax.experimental.pallas import tpu_sc as plsc

D = 128
DW = 16
CH = 128
NC = 2
NS = 16
NW = NC * NS
L = 16
GW = 20

_SC_PARAMS = pltpu.CompilerParams(
    use_tc_tiling_on_sc=False, needs_layout_passes=False)


def _msg_body(nf, w1, b1, w2, b2, out):
    h = jnp.maximum(jnp.dot(nf[...], w1[...], preferred_element_type=jnp.float32) + b1[...], 0.0)
    out[...] = jnp.dot(h, w2[...], preferred_element_type=jnp.float32) + b2[...]


def _sc_deg_body(nchunk, idxr, zr, degout, dst_v, w_v, deg_src, deg_sh, sem):
    cid = lax.axis_index("c")
    sid = lax.axis_index("s")
    wid = cid * NS + sid
    n_pad = deg_sh.shape[0]
    rpt = n_pad // NS

    pltpu.sync_copy(idxr.at[1, wid], dst_v)
    pltpu.sync_copy(idxr.at[2, wid], w_v)
    pltpu.sync_copy(zr.at[pl.ds(sid * rpt, rpt)], deg_sh.at[pl.ds(sid * rpt, rpt)])

    def zrow(i, c):
        deg_src[i, pl.ds(0, L)] = jnp.zeros((L,), jnp.float32)
        return c
    lax.fori_loop(0, CH, zrow, 0)
    plsc.subcore_barrier()

    col0 = jnp.zeros((L,), jnp.int32)
    iota = lax.broadcasted_iota(jnp.int32, (L,), 0)

    def chunk(j, carry):
        for g in range(CH // L):
            w16 = plsc.bitcast(w_v[j, pl.ds(g * L, L)], jnp.float32)
            plsc.store_scatter(deg_src, [g * L + iota, col0], w16)
        pltpu.sync_copy(deg_src, deg_sh.at[dst_v.at[j]], add=True)
        return carry

    lax.fori_loop(0, nchunk, chunk, 0)
    plsc.subcore_barrier()
    pltpu.sync_copy(deg_sh.at[pl.ds(sid * rpt, rpt)],
                    degout.at[cid, pl.ds(sid * rpt, rpt)])


def _sc_agg_body(nchunk, msgs, idxr, zr, aggout,
                 src_v, dst_v, w_v, rows_v, agg_sh, sem_g):
    cid = lax.axis_index("c")
    sid = lax.axis_index("s")
    wid = cid * NS + sid
    n_pad = agg_sh.shape[0]
    rpt = n_pad // NS

    pltpu.sync_copy(zr.at[pl.ds(sid * rpt, rpt)], agg_sh.at[pl.ds(sid * rpt, rpt)])
    plsc.subcore_barrier()

    pltpu.sync_copy(idxr.at[0, wid], src_v)
    pltpu.sync_copy(idxr.at[1, wid], dst_v)
    pltpu.sync_copy(idxr.at[2, wid], w_v)

    def chunk(j, c0):
        pltpu.async_copy(msgs.at[src_v.at[j]], rows_v, sem_g).wait()
        jv = jnp.full((L,), j, jnp.int32)

        def row4(i4, c2):
            for r in range(4):
                i = i4 * 4 + r
                ws = plsc.bitcast(plsc.load_gather(
                    w_v, [jv, jnp.full((L,), i, jnp.int32)]), jnp.float32)
                for c in range(D // L):
                    rows_v[i, pl.ds(c * L, L)] = rows_v[i, pl.ds(c * L, L)] * ws
            return c2

        lax.fori_loop(0, CH // 4, row4, 0)
        pltpu.sync_copy(rows_v, agg_sh.at[dst_v.at[j]], add=True)
        return c0

    lax.fori_loop(0, nchunk, chunk, 0)
    plsc.subcore_barrier()
    pltpu.sync_copy(agg_sh.at[pl.ds(sid * rpt, rpt)],
                    aggout.at[cid, pl.ds(sid * rpt, rpt)])


def _final_body(aggp, degp, nf, wga, wgn, bg, wua, wun, bu1, wu2, bu2, gamma, beta, out):
    p = aggp[...]
    dp = degp[...]
    deg = jnp.maximum((dp[0] + dp[1])[:, 0:1], 1.0)
    agg = (p[0] + p[1]) / deg
    x = nf[...]
    gate = jax.nn.sigmoid(
        jnp.dot(agg, wga[...], preferred_element_type=jnp.float32)
        + jnp.dot(x, wgn[...], preferred_element_type=jnp.float32) + bg[...])
    u = jnp.maximum(
        jnp.dot(agg, wua[...], preferred_element_type=jnp.float32)
        + jnp.dot(x, wun[...], preferred_element_type=jnp.float32) + bu1[...], 0.0)
    upd = jnp.dot(u, wu2[...], preferred_element_type=jnp.float32) + bu2[...]
    o = gate * upd + (1.0 - gate) * x
    mean = jnp.mean(o, axis=-1, keepdims=True)
    var = jnp.mean((o - mean) ** 2, axis=-1, keepdims=True)
    out[...] = (o - mean) / jnp.sqrt(var + 1e-3) * gamma[...] + beta[...]


def kernel(node_feats, neighbor_feats, edge_indices, edge_weights,
           W1, b1, W2, b2, Wg, bg, Wu1, bu1, Wu2, bu2, gamma, beta):
    n = node_feats.shape[0]
    e = edge_indices.shape[1]
    bn = 1000
    grid = (n // bn,)

    full = lambda i: (0, 0)
    row_spec = pl.BlockSpec((bn, D), lambda i: (i, 0))
    w_spec = pl.BlockSpec((D, D), full)
    b_spec = pl.BlockSpec((1, D), full)

    msgs = pl.pallas_call(
        _msg_body,
        grid=grid,
        in_specs=[row_spec, w_spec, b_spec, w_spec, b_spec],
        out_specs=row_spec,
        out_shape=jax.ShapeDtypeStruct((n, D), jnp.float32),
    )(neighbor_feats, W1, b1.reshape(1, D), W2, b2.reshape(1, D))

    epw = -(-e // (NW * CH)) * CH
    ep = epw * NW
    pad = ep - e
    src = jnp.concatenate([edge_indices[0], jnp.zeros((pad,), jnp.int32)])
    dst = jnp.concatenate([edge_indices[1], jnp.zeros((pad,), jnp.int32)])
    w = jnp.concatenate([edge_weights[:, 0], jnp.zeros((pad,), jnp.float32)])
    nchunk = epw // CH
    idx3 = jnp.stack([
        src.reshape(NW, nchunk, CH),
        dst.reshape(NW, nchunk, CH),
        lax.bitcast_convert_type(w, jnp.int32).reshape(NW, nchunk, CH),
    ])
    n_pad = -(-n // (NS * 8)) * (NS * 8)

    deg_partial = pl.kernel(
        lambda *refs: _sc_deg_body(nchunk, *refs),
        out_type=jax.ShapeDtypeStruct((NC, n_pad, DW), jnp.float32),
        mesh=plsc.VectorSubcoreMesh(
            core_axis_name="c", subcore_axis_name="s",
            num_cores=NC, num_subcores=NS),
        scratch_types=[
            pltpu.VMEM((nchunk, CH), jnp.int32),
            pltpu.VMEM((nchunk, CH), jnp.int32),
            pltpu.VMEM((CH, DW), jnp.float32),
            pltpu.VMEM_SHARED((n_pad, DW), jnp.float32),
            pltpu.SemaphoreType.DMA,
        ],
        compiler_params=_SC_PARAMS,
    )(idx3, jnp.zeros((n_pad, DW), jnp.float32))

    agg_partial = pl.kernel(
        lambda *refs: _sc_agg_body(nchunk, *refs),
        out_type=jax.ShapeDtypeStruct((NC, n_pad, D), jnp.float32),
        mesh=plsc.VectorSubcoreMesh(
            core_axis_name="c", subcore_axis_name="s",
            num_cores=NC, num_subcores=NS),
        scratch_types=[
            pltpu.VMEM((nchunk, CH), jnp.int32),
            pltpu.VMEM((nchunk, CH), jnp.int32),
            pltpu.VMEM((nchunk, CH), jnp.int32),
            pltpu.VMEM((CH, D), jnp.float32),
            pltpu.VMEM_SHARED((n_pad, D), jnp.float32),
            pltpu.SemaphoreType.DMA,
        ],
        compiler_params=_SC_PARAMS,
    )(msgs, idx3, jnp.zeros((n_pad, D), jnp.float32))

    out = pl.pallas_call(
        _final_body,
        grid=grid,
        in_specs=[pl.BlockSpec((NC, bn, D), lambda i: (0, i, 0)),
                  pl.BlockSpec((NC, bn, DW), lambda i: (0, i, 0)),
                  row_spec, w_spec, w_spec, b_spec,
                  w_spec, w_spec, b_spec, w_spec, b_spec, b_spec, b_spec],
        out_specs=row_spec,
        out_shape=jax.ShapeDtypeStruct((n, D), jnp.float32),
    )(agg_partial, deg_partial, node_feats, Wg[:D], Wg[D:], bg.reshape(1, D),
      Wu1[:D], Wu1[D:], bu1.reshape(1, D), Wu2, bu2.reshape(1, D),
      gamma.reshape(1, D), beta.reshape(1, D))
    return out
